# Initial kernel scaffold; baseline (speedup 1.0000x reference)
import functools

import jax
import jax.numpy as jnp
from jax import lax
from jax.experimental import pallas as pl
from jax.experimental.pallas import tpu as pltpu
import jax.experimental.pallas.tpu_sc as plsc

N = 10000
E = 320000
P = 100
D = 128
N_GNN = 7
N_EVAL = 5

N_PAD = 10240
E_PAD = 321536
CHUNK = 128
N_CHUNKS = E_PAD // (16 * CHUNK)
ROWS_PER_TILE = N_PAD // 16
LCHUNK = 64
L_CHUNKS = E_PAD // (32 * LCHUNK)
X_ROWS = N_PAD // 32
HI = lax.Precision.HIGHEST

_MESH = plsc.VectorSubcoreMesh(core_axis_name="c", subcore_axis_name="s")
_f32 = jnp.float32


@functools.partial(
    pl.kernel,
    out_type=[
        jax.ShapeDtypeStruct((N_PAD, 16), _f32),
        jax.ShapeDtypeStruct((N_PAD, 16), _f32),
        jax.ShapeDtypeStruct((N_PAD, 16), _f32),
        jax.ShapeDtypeStruct((N_PAD, 16), _f32),
        jax.ShapeDtypeStruct((N_PAD, D), _f32),
    ],
    mesh=_MESH,
    scratch_types=[
        pltpu.VMEM_SHARED((N_PAD, 16), _f32),
        pltpu.VMEM_SHARED((N_PAD, 16), _f32),
        pltpu.VMEM((CHUNK, 16), _f32),
        pltpu.VMEM((CHUNK, 16), _f32),
        pltpu.VMEM((CHUNK,), jnp.int32),
        pltpu.VMEM((LCHUNK,), jnp.int32),
        pltpu.VMEM((LCHUNK, D), _f32),
        pltpu.SemaphoreType.DMA,
    ],
)
def _sc_prep(nodes, s1, r1, s2, r2, embed,
             deg_s1, deg_r1, deg_s2, deg_r2, x0,
             acc_a, acc_b, ones_v, zeros_v, idx_v, nidx_v, xrows_v, sem):
    c = lax.axis_index("c")
    tile = lax.axis_index("s")

    def fill(i, _):
        ones_v[i, :] = jnp.ones((16,), _f32)
        zeros_v[i, :] = jnp.zeros((16,), _f32)
        return 0
    lax.fori_loop(0, CHUNK, fill, 0)

    base_r = tile * ROWS_PER_TILE
    for k in range(ROWS_PER_TILE // CHUNK):
        pltpu.sync_copy(zeros_v, acc_a.at[pl.ds(base_r + k * CHUNK, CHUNK)])
        pltpu.sync_copy(zeros_v, acc_b.at[pl.ds(base_r + k * CHUNK, CHUNK)])
    plsc.subcore_barrier()

    def deg_pass(sref, rref):
        def body(j, _):
            ebase = tile * (N_CHUNKS * CHUNK) + j * CHUNK
            pltpu.sync_copy(sref.at[pl.ds(ebase, CHUNK)], idx_v)
            pltpu.sync_copy(ones_v, acc_a.at[idx_v], add=True)
            pltpu.sync_copy(rref.at[pl.ds(ebase, CHUNK)], idx_v)
            pltpu.sync_copy(ones_v, acc_b.at[idx_v], add=True)
            return 0
        lax.fori_loop(0, N_CHUNKS, body, 0)

    @pl.when(c == 0)
    def _():
        deg_pass(s1, r1)

    @pl.when(c == 1)
    def _():
        deg_pass(s2, r2)

    plsc.subcore_barrier()

    @pl.when(c == 0)
    def _():
        pltpu.sync_copy(acc_a.at[pl.ds(base_r, ROWS_PER_TILE)],
                        deg_s1.at[pl.ds(base_r, ROWS_PER_TILE)])
        pltpu.sync_copy(acc_b.at[pl.ds(base_r, ROWS_PER_TILE)],
                        deg_r1.at[pl.ds(base_r, ROWS_PER_TILE)])

    @pl.when(c == 1)
    def _():
        pltpu.sync_copy(acc_a.at[pl.ds(base_r, ROWS_PER_TILE)],
                        deg_s2.at[pl.ds(base_r, ROWS_PER_TILE)])
        pltpu.sync_copy(acc_b.at[pl.ds(base_r, ROWS_PER_TILE)],
                        deg_r2.at[pl.ds(base_r, ROWS_PER_TILE)])

    w = tile * 2 + c
    for j in range(X_ROWS // LCHUNK):
        nbase = w * X_ROWS + j * LCHUNK
        pltpu.sync_copy(nodes.at[pl.ds(nbase, LCHUNK)], nidx_v)
        pltpu.async_copy(embed.at[nidx_v], xrows_v, sem).wait()
        pltpu.sync_copy(xrows_v, x0.at[pl.ds(nbase, LCHUNK)])


@functools.partial(
    pl.kernel,
    out_type=[
        jax.ShapeDtypeStruct((N_PAD, D), _f32),
        jax.ShapeDtypeStruct((N_PAD, D), _f32),
    ],
    mesh=_MESH,
    scratch_types=[
        pltpu.VMEM_SHARED((N_PAD, D), _f32),
        pltpu.VMEM((CHUNK, D), _f32),
        pltpu.VMEM((CHUNK, D), _f32),
        pltpu.VMEM((CHUNK,), jnp.int32),
        pltpu.VMEM((CHUNK,), jnp.int32),
        pltpu.SemaphoreType.DMA,
    ],
)
def _sc_agg(h1, h2, s1, r1, s2, r2, o1, o2,
            acc, rows_v, zeros_v, sidx_v, ridx_v, sem):
    c = lax.axis_index("c")
    tile = lax.axis_index("s")

    def fill(i, _):
        for t in range(D // 16):
            zeros_v[i, pl.ds(t * 16, 16)] = jnp.zeros((16,), _f32)
        return 0
    lax.fori_loop(0, CHUNK, fill, 0)

    base_r = tile * ROWS_PER_TILE
    for k in range(ROWS_PER_TILE // CHUNK):
        pltpu.sync_copy(zeros_v, acc.at[pl.ds(base_r + k * CHUNK, CHUNK)])
    plsc.subcore_barrier()

    def agg(href, sref, rref):
        def body(j, _):
            ebase = tile * (N_CHUNKS * CHUNK) + j * CHUNK
            pltpu.sync_copy(sref.at[pl.ds(ebase, CHUNK)], sidx_v)
            cp = pltpu.async_copy(href.at[sidx_v], rows_v, sem)
            pltpu.sync_copy(rref.at[pl.ds(ebase, CHUNK)], ridx_v)
            cp.wait()
            pltpu.sync_copy(rows_v, acc.at[ridx_v], add=True)
            return 0
        lax.fori_loop(0, N_CHUNKS, body, 0)

    @pl.when(c == 0)
    def _():
        agg(h1, s1, r1)

    @pl.when(c == 1)
    def _():
        agg(h2, s2, r2)

    plsc.subcore_barrier()

    @pl.when(c == 0)
    def _():
        pltpu.sync_copy(acc.at[pl.ds(base_r, ROWS_PER_TILE)],
                        o1.at[pl.ds(base_r, ROWS_PER_TILE)])

    @pl.when(c == 1)
    def _():
        pltpu.sync_copy(acc.at[pl.ds(base_r, ROWS_PER_TILE)],
                        o2.at[pl.ds(base_r, ROWS_PER_TILE)])


@functools.partial(
    pl.kernel,
    out_type=jax.ShapeDtypeStruct((E_PAD, 16), _f32),
    mesh=_MESH,
    scratch_types=[
        pltpu.VMEM((LCHUNK,), jnp.int32),
        pltpu.VMEM((LCHUNK,), jnp.int32),
        pltpu.VMEM((LCHUNK, D), _f32),
        pltpu.VMEM((LCHUNK, D), _f32),
        pltpu.VMEM((LCHUNK, 16), _f32),
        pltpu.SemaphoreType.DMA,
        pltpu.SemaphoreType.DMA,
    ],
)
def _sc_logits(ln, s1, r1, out, sidx_v, ridx_v, rs_v, rr_v, ov_v, sem_s, sem_r):
    c = lax.axis_index("c")
    tile = lax.axis_index("s")
    w = tile * 2 + c

    def body(j, _):
        ebase = w * (L_CHUNKS * LCHUNK) + j * LCHUNK
        pltpu.sync_copy(s1.at[pl.ds(ebase, LCHUNK)], sidx_v)
        cps = pltpu.async_copy(ln.at[sidx_v], rs_v, sem_s)
        pltpu.sync_copy(r1.at[pl.ds(ebase, LCHUNK)], ridx_v)
        cpr = pltpu.async_copy(ln.at[ridx_v], rr_v, sem_r)
        cps.wait()
        cpr.wait()

        def row(i, _):
            acc = rs_v[i, pl.ds(0, 16)] * rr_v[i, pl.ds(0, 16)]
            for t in range(1, D // 16):
                acc = acc + rs_v[i, pl.ds(t * 16, 16)] * rr_v[i, pl.ds(t * 16, 16)]
            ov_v[i, :] = acc
            return 0
        lax.fori_loop(0, LCHUNK, row, 0)
        pltpu.sync_copy(ov_v, out.at[pl.ds(ebase, LCHUNK)])
        return 0
    lax.fori_loop(0, L_CHUNKS, body, 0)


def _tc_layer0_body(x0_ref, ds1_ref, dr1_ref, ds2_ref, dr2_ref,
                    W1_ref, b1_ref, W2_ref, b2_ref,
                    h1_ref, h2_ref, is1_ref, is2_ref, ir1_ref, ir2_ref):
    x0 = x0_ref[...]
    shape = x0.shape
    is1 = jnp.broadcast_to(lax.rsqrt(ds1_ref[:, :1] + 1.0), shape)
    is2 = jnp.broadcast_to(lax.rsqrt(ds2_ref[:, :1] + 1.0), shape)
    ir1 = jnp.broadcast_to(lax.rsqrt(dr1_ref[:, :1] + 1.0), shape)
    ir2 = jnp.broadcast_to(lax.rsqrt(dr2_ref[:, :1] + 1.0), shape)
    h1_ref[...] = (jnp.dot(x0, W1_ref[...], precision=HI) + b1_ref[...]) * is1
    h2_ref[...] = (jnp.dot(x0, W2_ref[...], precision=HI) + b2_ref[...]) * is2
    is1_ref[...] = is1
    is2_ref[...] = is2
    ir1_ref[...] = ir1
    ir2_ref[...] = ir2


_BLK = 512
_GRID = N_PAD // _BLK


def _rows(shape=(_BLK, D)):
    return pl.BlockSpec(shape, lambda i: (i, 0))


def _full(shape):
    return pl.BlockSpec(shape, lambda i: (0, 0))


_tc_layer0 = pl.pallas_call(
    _tc_layer0_body,
    grid=(_GRID,),
    in_specs=[_rows(), _rows((_BLK, 16)), _rows((_BLK, 16)), _rows((_BLK, 16)),
              _rows((_BLK, 16)), _full((D, D)), _full((1, D)), _full((D, D)),
              _full((1, D))],
    out_specs=[_rows(), _rows(), _rows(), _rows(), _rows(), _rows()],
    out_shape=[jax.ShapeDtypeStruct((N_PAD, D), _f32)] * 6,
)


def _tc_layer_body(o1_ref, o2_ref, h1_ref, h2_ref, ir1_ref, ir2_ref,
                   is1_ref, is2_ref, WmA_ref, WmG_ref, bm_ref,
                   W1n_ref, b1n_ref, W2n_ref, b2n_ref,
                   h1n_ref, h2n_ref):
    a = (o1_ref[...] + h1_ref[...]) * ir1_ref[...]
    g = (o2_ref[...] + h2_ref[...]) * ir2_ref[...]
    xn = jnp.maximum(jnp.dot(a, WmA_ref[...], precision=HI)
                     + jnp.dot(g, WmG_ref[...], precision=HI) + bm_ref[...], 0.0)
    h1n_ref[...] = (jnp.dot(xn, W1n_ref[...], precision=HI) + b1n_ref[...]) * is1_ref[...]
    h2n_ref[...] = (jnp.dot(xn, W2n_ref[...], precision=HI) + b2n_ref[...]) * is2_ref[...]


_tc_layer = pl.pallas_call(
    _tc_layer_body,
    grid=(_GRID,),
    in_specs=[_rows()] * 8 + [_full((D, D)), _full((D, D)), _full((1, D)),
                              _full((D, D)), _full((1, D)), _full((D, D)),
                              _full((1, D))],
    out_specs=[_rows(), _rows()],
    out_shape=[jax.ShapeDtypeStruct((N_PAD, D), _f32)] * 2,
)


def _tc_final_body(o1_ref, o2_ref, h1_ref, h2_ref, ir1_ref, ir2_ref,
                   WmA_ref, WmG_ref, bm_ref, Wl_ref, bl_ref,
                   x_ref, ln_ref):
    a = (o1_ref[...] + h1_ref[...]) * ir1_ref[...]
    g = (o2_ref[...] + h2_ref[...]) * ir2_ref[...]
    xn = jnp.maximum(jnp.dot(a, WmA_ref[...], precision=HI)
                     + jnp.dot(g, WmG_ref[...], precision=HI) + bm_ref[...], 0.0)
    x_ref[...] = xn
    ln_ref[...] = jnp.dot(xn, Wl_ref[...], precision=HI) + bl_ref[...]


_tc_final = pl.pallas_call(
    _tc_final_body,
    grid=(_GRID,),
    in_specs=[_rows()] * 6 + [_full((D, D)), _full((D, D)), _full((1, D)),
                              _full((D, D)), _full((1, D))],
    out_specs=[_rows(), _rows()],
    out_shape=[jax.ShapeDtypeStruct((N_PAD, D), _f32)] * 2,
)


def _tc_eval_body(x_ref, We_ref, be_ref, Wo_ref, bo_ref, v_ref):
    x = x_ref[...]
    pid = lax.broadcasted_iota(jnp.int32, (D, N_PAD), 0)
    nid = lax.broadcasted_iota(jnp.int32, (D, N_PAD), 1)
    seg = (nid // (N // P) == pid) & (nid % (N // P) != 0)
    v = jnp.dot(seg.astype(_f32), x, precision=HI) * (1.0 / (N // P - 1))
    We = We_ref[...]
    be = be_ref[...]
    for i in range(N_EVAL):
        v = jnp.maximum(
            jnp.dot(v, We[i * D:(i + 1) * D, :], precision=HI) + be[i:i + 1, :],
            0.0)
    v_ref[...] = jnp.tanh(jnp.dot(v, Wo_ref[...], precision=HI) + bo_ref[...])


_tc_eval = pl.pallas_call(
    _tc_eval_body,
    out_shape=jax.ShapeDtypeStruct((D, D), _f32),
)


def _tc_lsum_body(pv_ref, out_ref):
    out_ref[...] = jnp.sum(pv_ref[...], axis=1, keepdims=True)


_tc_lsum = pl.pallas_call(
    _tc_lsum_body,
    grid=(128,),
    in_specs=[pl.BlockSpec((E_PAD // 128, 16), lambda i: (i, 0))],
    out_specs=pl.BlockSpec((E_PAD // 128, 1), lambda i: (i, 0)),
    out_shape=jax.ShapeDtypeStruct((E_PAD, 1), _f32),
)


def kernel(nodes, senders, receivers, grid_senders, grid_receivers, n_node,
           embed, W_conv1, b_conv1, W_conv2, b_conv2, W_mix, b_mix,
           W_logits, b_logits, W_eval, b_eval, W_out, b_out):
    pad_e = jnp.full((E_PAD - E,), N, jnp.int32)
    s1 = jnp.concatenate([senders, pad_e])
    r1 = jnp.concatenate([receivers, pad_e])
    s2 = jnp.concatenate([grid_senders, pad_e])
    r2 = jnp.concatenate([grid_receivers, pad_e])
    nodes_p = jnp.concatenate([nodes, jnp.zeros((N_PAD - N,), jnp.int32)])

    deg_s1, deg_r1, deg_s2, deg_r2, x0 = _sc_prep(nodes_p, s1, r1, s2, r2, embed)

    h1, h2, is1, is2, ir1, ir2 = _tc_layer0(
        x0, deg_s1, deg_r1, deg_s2, deg_r2,
        W_conv1[0], b_conv1[0].reshape(1, D), W_conv2[0], b_conv2[0].reshape(1, D))

    for i in range(N_GNN - 1):
        o1, o2 = _sc_agg(h1, h2, s1, r1, s2, r2)
        h1, h2 = _tc_layer(
            o1, o2, h1, h2, ir1, ir2, is1, is2,
            W_mix[i, :D, :], W_mix[i, D:, :], b_mix[i].reshape(1, D),
            W_conv1[i + 1], b_conv1[i + 1].reshape(1, D),
            W_conv2[i + 1], b_conv2[i + 1].reshape(1, D))

    o1, o2 = _sc_agg(h1, h2, s1, r1, s2, r2)
    x, ln = _tc_final(
        o1, o2, h1, h2, ir1, ir2,
        W_mix[6, :D, :], W_mix[6, D:, :], b_mix[6].reshape(1, D),
        W_logits, b_logits.reshape(1, D))

    pv = _sc_logits(ln, s1, r1)
    logits = _tc_lsum(pv)[:E, 0]

    v = _tc_eval(x, W_eval.reshape(N_EVAL * D, D), b_eval,
                 jnp.pad(W_out, ((0, 0), (0, D - 1))),
                 jnp.pad(b_out.reshape(1, 1), ((0, 0), (0, D - 1))))
    return logits, v[:P, :1]


# speedup vs baseline: 6.0134x; 6.0134x over previous
import functools

import jax
import jax.numpy as jnp
from jax import lax
from jax.experimental import pallas as pl
from jax.experimental.pallas import tpu as pltpu
import jax.experimental.pallas.tpu_sc as plsc

N = 10000
E = 320000
P = 100
D = 128
N_GNN = 7
N_EVAL = 5

N_PAD = 10240
E_PAD = 321536
CHUNK = 128
N_CHUNKS = E_PAD // (16 * CHUNK)
ROWS_PER_TILE = N_PAD // 16
LCHUNK = 64
L_CHUNKS = E_PAD // (32 * LCHUNK)
X_ROWS = N_PAD // 32
HI = lax.Precision.HIGHEST

_f32 = jnp.float32


@functools.cache
def _sc_kernels():
    mesh = plsc.VectorSubcoreMesh(core_axis_name="c", subcore_axis_name="s",
                                  num_cores=2, num_subcores=16)

    @functools.partial(
        pl.kernel,
        out_type=[
            jax.ShapeDtypeStruct((N_PAD, 16), _f32),
            jax.ShapeDtypeStruct((N_PAD, 16), _f32),
            jax.ShapeDtypeStruct((N_PAD, 16), _f32),
            jax.ShapeDtypeStruct((N_PAD, 16), _f32),
            jax.ShapeDtypeStruct((N_PAD, D), _f32),
        ],
        mesh=mesh,
        scratch_types=[
            pltpu.VMEM_SHARED((N_PAD, 16), _f32),
            pltpu.VMEM_SHARED((N_PAD, 16), _f32),
            pltpu.VMEM((CHUNK, 16), _f32),
            pltpu.VMEM((CHUNK, 16), _f32),
            pltpu.VMEM((CHUNK,), jnp.int32),
            pltpu.VMEM((LCHUNK,), jnp.int32),
            pltpu.VMEM((LCHUNK, D), _f32),
            pltpu.SemaphoreType.DMA,
        ],
    )
    def _sc_prep(nodes, s1, r1, s2, r2, embed,
                 deg_s1, deg_r1, deg_s2, deg_r2, x0,
                 acc_a, acc_b, ones_v, zeros_v, idx_v, nidx_v, xrows_v, sem):
        c = lax.axis_index("c")
        tile = lax.axis_index("s")

        def fill(i, _):
            ones_v[i, :] = jnp.ones((16,), _f32)
            zeros_v[i, :] = jnp.zeros((16,), _f32)
            return 0
        lax.fori_loop(0, CHUNK, fill, 0)

        base_r = tile * ROWS_PER_TILE
        for k in range(ROWS_PER_TILE // CHUNK):
            pltpu.sync_copy(zeros_v, acc_a.at[pl.ds(base_r + k * CHUNK, CHUNK)])
            pltpu.sync_copy(zeros_v, acc_b.at[pl.ds(base_r + k * CHUNK, CHUNK)])
        plsc.subcore_barrier()

        def deg_pass(sref, rref):
            def body(j, _):
                ebase = tile * (N_CHUNKS * CHUNK) + j * CHUNK
                pltpu.sync_copy(sref.at[pl.ds(ebase, CHUNK)], idx_v)
                pltpu.sync_copy(ones_v, acc_a.at[idx_v], add=True)
                pltpu.sync_copy(rref.at[pl.ds(ebase, CHUNK)], idx_v)
                pltpu.sync_copy(ones_v, acc_b.at[idx_v], add=True)
                return 0
            lax.fori_loop(0, N_CHUNKS, body, 0)

        @pl.when(c == 0)
        def _():
            deg_pass(s1, r1)

        @pl.when(c == 1)
        def _():
            deg_pass(s2, r2)

        plsc.subcore_barrier()

        @pl.when(c == 0)
        def _():
            pltpu.sync_copy(acc_a.at[pl.ds(base_r, ROWS_PER_TILE)],
                            deg_s1.at[pl.ds(base_r, ROWS_PER_TILE)])
            pltpu.sync_copy(acc_b.at[pl.ds(base_r, ROWS_PER_TILE)],
                            deg_r1.at[pl.ds(base_r, ROWS_PER_TILE)])

        @pl.when(c == 1)
        def _():
            pltpu.sync_copy(acc_a.at[pl.ds(base_r, ROWS_PER_TILE)],
                            deg_s2.at[pl.ds(base_r, ROWS_PER_TILE)])
            pltpu.sync_copy(acc_b.at[pl.ds(base_r, ROWS_PER_TILE)],
                            deg_r2.at[pl.ds(base_r, ROWS_PER_TILE)])

        w = tile * 2 + c
        for j in range(X_ROWS // LCHUNK):
            nbase = w * X_ROWS + j * LCHUNK
            pltpu.sync_copy(nodes.at[pl.ds(nbase, LCHUNK)], nidx_v)
            pltpu.async_copy(embed.at[nidx_v], xrows_v, sem).wait()
            pltpu.sync_copy(xrows_v, x0.at[pl.ds(nbase, LCHUNK)])

    @functools.partial(
        pl.kernel,
        out_type=[
            jax.ShapeDtypeStruct((N_PAD, D), _f32),
            jax.ShapeDtypeStruct((N_PAD, D), _f32),
        ],
        mesh=mesh,
        scratch_types=[
            pltpu.VMEM_SHARED((N_PAD, D), _f32),
            pltpu.VMEM((CHUNK, D), _f32),
            pltpu.VMEM((CHUNK, D), _f32),
            pltpu.VMEM((CHUNK,), jnp.int32),
            pltpu.VMEM((CHUNK,), jnp.int32),
            pltpu.SemaphoreType.DMA,
        ],
    )
    def _sc_agg(h1, h2, s1, r1, s2, r2, o1, o2,
                acc, rows_v, zeros_v, sidx_v, ridx_v, sem):
        c = lax.axis_index("c")
        tile = lax.axis_index("s")

        def fill(i, _):
            for t in range(D // 16):
                zeros_v[i, pl.ds(t * 16, 16)] = jnp.zeros((16,), _f32)
            return 0
        lax.fori_loop(0, CHUNK, fill, 0)

        base_r = tile * ROWS_PER_TILE
        for k in range(ROWS_PER_TILE // CHUNK):
            pltpu.sync_copy(zeros_v, acc.at[pl.ds(base_r + k * CHUNK, CHUNK)])
        plsc.subcore_barrier()

        def agg(href, sref, rref):
            def body(j, _):
                ebase = tile * (N_CHUNKS * CHUNK) + j * CHUNK
                pltpu.sync_copy(sref.at[pl.ds(ebase, CHUNK)], sidx_v)
                cp = pltpu.async_copy(href.at[sidx_v], rows_v, sem)
                pltpu.sync_copy(rref.at[pl.ds(ebase, CHUNK)], ridx_v)
                cp.wait()
                pltpu.sync_copy(rows_v, acc.at[ridx_v], add=True)
                return 0
            lax.fori_loop(0, N_CHUNKS, body, 0)

        @pl.when(c == 0)
        def _():
            agg(h1, s1, r1)

        @pl.when(c == 1)
        def _():
            agg(h2, s2, r2)

        plsc.subcore_barrier()

        @pl.when(c == 0)
        def _():
            pltpu.sync_copy(acc.at[pl.ds(base_r, ROWS_PER_TILE)],
                            o1.at[pl.ds(base_r, ROWS_PER_TILE)])

        @pl.when(c == 1)
        def _():
            pltpu.sync_copy(acc.at[pl.ds(base_r, ROWS_PER_TILE)],
                            o2.at[pl.ds(base_r, ROWS_PER_TILE)])

    @functools.partial(
        pl.kernel,
        out_type=jax.ShapeDtypeStruct((E_PAD, 16), _f32),
        mesh=mesh,
        scratch_types=[
            pltpu.VMEM((LCHUNK,), jnp.int32),
            pltpu.VMEM((LCHUNK,), jnp.int32),
            pltpu.VMEM((LCHUNK, D), _f32),
            pltpu.VMEM((LCHUNK, D), _f32),
            pltpu.VMEM((LCHUNK, 16), _f32),
            pltpu.SemaphoreType.DMA,
            pltpu.SemaphoreType.DMA,
        ],
    )
    def _sc_logits(ln, s1, r1, out,
                   sidx_v, ridx_v, rs_v, rr_v, ov_v, sem_s, sem_r):
        c = lax.axis_index("c")
        tile = lax.axis_index("s")
        w = tile * 2 + c

        def body(j, _):
            ebase = w * (L_CHUNKS * LCHUNK) + j * LCHUNK
            pltpu.sync_copy(s1.at[pl.ds(ebase, LCHUNK)], sidx_v)
            cps = pltpu.async_copy(ln.at[sidx_v], rs_v, sem_s)
            pltpu.sync_copy(r1.at[pl.ds(ebase, LCHUNK)], ridx_v)
            cpr = pltpu.async_copy(ln.at[ridx_v], rr_v, sem_r)
            cps.wait()
            cpr.wait()

            def row(i, _):
                acc = rs_v[i, pl.ds(0, 16)] * rr_v[i, pl.ds(0, 16)]
                for t in range(1, D // 16):
                    acc = acc + rs_v[i, pl.ds(t * 16, 16)] * rr_v[i, pl.ds(t * 16, 16)]
                ov_v[i, :] = acc
                return 0
            lax.fori_loop(0, LCHUNK, row, 0)
            pltpu.sync_copy(ov_v, out.at[pl.ds(ebase, LCHUNK)])
            return 0
        lax.fori_loop(0, L_CHUNKS, body, 0)

    return _sc_prep, _sc_agg, _sc_logits


def _tc_layer0_body(x0_ref, ds1_ref, dr1_ref, ds2_ref, dr2_ref,
                    W1_ref, b1_ref, W2_ref, b2_ref,
                    h1_ref, h2_ref, is1_ref, is2_ref, ir1_ref, ir2_ref):
    x0 = x0_ref[...]
    shape = x0.shape
    is1 = jnp.broadcast_to(lax.rsqrt(ds1_ref[:, :1] + 1.0), shape)
    is2 = jnp.broadcast_to(lax.rsqrt(ds2_ref[:, :1] + 1.0), shape)
    ir1 = jnp.broadcast_to(lax.rsqrt(dr1_ref[:, :1] + 1.0), shape)
    ir2 = jnp.broadcast_to(lax.rsqrt(dr2_ref[:, :1] + 1.0), shape)
    h1_ref[...] = (jnp.dot(x0, W1_ref[...], precision=HI) + b1_ref[...]) * is1
    h2_ref[...] = (jnp.dot(x0, W2_ref[...], precision=HI) + b2_ref[...]) * is2
    is1_ref[...] = is1
    is2_ref[...] = is2
    ir1_ref[...] = ir1
    ir2_ref[...] = ir2


_BLK = 512
_GRID = N_PAD // _BLK


def _rows(shape=(_BLK, D)):
    return pl.BlockSpec(shape, lambda i: (i, 0))


def _full(shape):
    return pl.BlockSpec(shape, lambda i: (0, 0))


_tc_layer0 = pl.pallas_call(
    _tc_layer0_body,
    grid=(_GRID,),
    in_specs=[_rows(), _rows((_BLK, 16)), _rows((_BLK, 16)), _rows((_BLK, 16)),
              _rows((_BLK, 16)), _full((D, D)), _full((1, D)), _full((D, D)),
              _full((1, D))],
    out_specs=[_rows(), _rows(), _rows(), _rows(), _rows(), _rows()],
    out_shape=[jax.ShapeDtypeStruct((N_PAD, D), _f32)] * 6,
)


def _tc_layer_body(o1_ref, o2_ref, h1_ref, h2_ref, ir1_ref, ir2_ref,
                   is1_ref, is2_ref, WmA_ref, WmG_ref, bm_ref,
                   W1n_ref, b1n_ref, W2n_ref, b2n_ref,
                   h1n_ref, h2n_ref):
    a = (o1_ref[...] + h1_ref[...]) * ir1_ref[...]
    g = (o2_ref[...] + h2_ref[...]) * ir2_ref[...]
    xn = jnp.maximum(jnp.dot(a, WmA_ref[...], precision=HI)
                     + jnp.dot(g, WmG_ref[...], precision=HI) + bm_ref[...], 0.0)
    h1n_ref[...] = (jnp.dot(xn, W1n_ref[...], precision=HI) + b1n_ref[...]) * is1_ref[...]
    h2n_ref[...] = (jnp.dot(xn, W2n_ref[...], precision=HI) + b2n_ref[...]) * is2_ref[...]


_tc_layer = pl.pallas_call(
    _tc_layer_body,
    grid=(_GRID,),
    in_specs=[_rows()] * 8 + [_full((D, D)), _full((D, D)), _full((1, D)),
                              _full((D, D)), _full((1, D)), _full((D, D)),
                              _full((1, D))],
    out_specs=[_rows(), _rows()],
    out_shape=[jax.ShapeDtypeStruct((N_PAD, D), _f32)] * 2,
)


def _tc_final_body(o1_ref, o2_ref, h1_ref, h2_ref, ir1_ref, ir2_ref,
                   WmA_ref, WmG_ref, bm_ref, Wl_ref, bl_ref,
                   x_ref, ln_ref):
    a = (o1_ref[...] + h1_ref[...]) * ir1_ref[...]
    g = (o2_ref[...] + h2_ref[...]) * ir2_ref[...]
    xn = jnp.maximum(jnp.dot(a, WmA_ref[...], precision=HI)
                     + jnp.dot(g, WmG_ref[...], precision=HI) + bm_ref[...], 0.0)
    x_ref[...] = xn
    ln_ref[...] = jnp.dot(xn, Wl_ref[...], precision=HI) + bl_ref[...]


_tc_final = pl.pallas_call(
    _tc_final_body,
    grid=(_GRID,),
    in_specs=[_rows()] * 6 + [_full((D, D)), _full((D, D)), _full((1, D)),
                              _full((D, D)), _full((1, D))],
    out_specs=[_rows(), _rows()],
    out_shape=[jax.ShapeDtypeStruct((N_PAD, D), _f32)] * 2,
)


def _tc_eval_body(x_ref, We_ref, be_ref, Wo_ref, bo_ref, v_ref):
    x = x_ref[...]
    pid = lax.broadcasted_iota(jnp.int32, (D, N_PAD), 0)
    nid = lax.broadcasted_iota(jnp.int32, (D, N_PAD), 1)
    seg = (nid // (N // P) == pid) & (nid % (N // P) != 0)
    v = jnp.dot(seg.astype(_f32), x, precision=HI) * (1.0 / (N // P - 1))
    We = We_ref[...]
    be = be_ref[...]
    for i in range(N_EVAL):
        v = jnp.maximum(
            jnp.dot(v, We[i * D:(i + 1) * D, :], precision=HI) + be[i:i + 1, :],
            0.0)
    v_ref[...] = jnp.tanh(jnp.dot(v, Wo_ref[...], precision=HI) + bo_ref[...])


_tc_eval = pl.pallas_call(
    _tc_eval_body,
    out_shape=jax.ShapeDtypeStruct((D, D), _f32),
)


def _tc_lsum_body(pv_ref, out_ref):
    out_ref[...] = jnp.sum(pv_ref[...], axis=1, keepdims=True)


_tc_lsum = pl.pallas_call(
    _tc_lsum_body,
    grid=(128,),
    in_specs=[pl.BlockSpec((E_PAD // 128, 16), lambda i: (i, 0))],
    out_specs=pl.BlockSpec((E_PAD // 128, 1), lambda i: (i, 0)),
    out_shape=jax.ShapeDtypeStruct((E_PAD, 1), _f32),
)


def kernel(nodes, senders, receivers, grid_senders, grid_receivers, n_node,
           embed, W_conv1, b_conv1, W_conv2, b_conv2, W_mix, b_mix,
           W_logits, b_logits, W_eval, b_eval, W_out, b_out):
    sc_prep, sc_agg, sc_logits = _sc_kernels()

    pad_e = jnp.full((E_PAD - E,), N, jnp.int32)
    s1 = jnp.concatenate([senders, pad_e])
    r1 = jnp.concatenate([receivers, pad_e])
    s2 = jnp.concatenate([grid_senders, pad_e])
    r2 = jnp.concatenate([grid_receivers, pad_e])
    nodes_p = jnp.concatenate([nodes, jnp.zeros((N_PAD - N,), jnp.int32)])

    deg_s1, deg_r1, deg_s2, deg_r2, x0 = sc_prep(nodes_p, s1, r1, s2, r2, embed)

    h1, h2, is1, is2, ir1, ir2 = _tc_layer0(
        x0, deg_s1, deg_r1, deg_s2, deg_r2,
        W_conv1[0], b_conv1[0].reshape(1, D), W_conv2[0], b_conv2[0].reshape(1, D))

    for i in range(N_GNN - 1):
        o1, o2 = sc_agg(h1, h2, s1, r1, s2, r2)
        h1, h2 = _tc_layer(
            o1, o2, h1, h2, ir1, ir2, is1, is2,
            W_mix[i, :D, :], W_mix[i, D:, :], b_mix[i].reshape(1, D),
            W_conv1[i + 1], b_conv1[i + 1].reshape(1, D),
            W_conv2[i + 1], b_conv2[i + 1].reshape(1, D))

    o1, o2 = sc_agg(h1, h2, s1, r1, s2, r2)
    x, ln = _tc_final(
        o1, o2, h1, h2, ir1, ir2,
        W_mix[6, :D, :], W_mix[6, D:, :], b_mix[6].reshape(1, D),
        W_logits, b_logits.reshape(1, D))

    pv = sc_logits(ln, s1, r1)
    logits = _tc_lsum(pv)[:E, 0]

    v = _tc_eval(x, W_eval.reshape(N_EVAL * D, D), b_eval,
                 jnp.pad(W_out, ((0, 0), (0, D - 1))),
                 jnp.pad(b_out.reshape(1, 1), ((0, 0), (0, D - 1))))
    return logits, v[:P, :1]
